# baseline (speedup 1.0000x reference)
import jax
import jax.numpy as jnp
from jax import lax
from jax.experimental import pallas as pl
from jax.experimental.pallas import tpu as pltpu
from jax.experimental.pallas import tpu_sc as plsc

_N_VOCAB = 1000000
_N_EMBED = 64
_BATCH = 4096
_HIST = 50

_N_TOT = _BATCH * _HIST
_NW = 32
_PER_W = _N_TOT // _NW
_CHUNK = 128
_NCH = _PER_W // _CHUNK
_LANES = 16


def _emb_body(x_hbm, w_hbm, out_hbm, idx_v, rows_v, gsem):
    wid = lax.axis_index("s") * 2 + lax.axis_index("c")
    pltpu.sync_copy(x_hbm.at[wid], idx_v)

    def chunk_body(j, carry):
        pltpu.async_copy(w_hbm.at[idx_v.at[j]], rows_v, gsem).wait()
        zeros = jnp.zeros((_LANES,), jnp.float32)
        for g in range(_CHUNK // _LANES):
            idxs = idx_v[j, pl.ds(g * _LANES, _LANES)]
            msk = idxs == 0
            cnt = plsc.all_reduce_population_count(msk)

            @pl.when(cnt[0] > 0)
            def _():
                rowpos = g * _LANES + lax.iota(jnp.int32, 16)
                for c in range(_N_EMBED):
                    colpos = jnp.full((_LANES,), c, jnp.int32)
                    plsc.store_scatter(rows_v, [rowpos, colpos], zeros,
                                       mask=msk)

        row_base = wid * _PER_W + j * _CHUNK
        pltpu.sync_copy(rows_v, out_hbm.at[pl.ds(row_base, _CHUNK)])
        return carry

    lax.fori_loop(0, _NCH, chunk_body, 0)


@jax.jit
def kernel(x, W):
    x_flat = x.reshape(_NW, _NCH, _CHUNK)
    call = pl.kernel(
        _emb_body,
        out_type=jax.ShapeDtypeStruct((_N_TOT, _N_EMBED), jnp.float32),
        mesh=plsc.VectorSubcoreMesh(core_axis_name="c", subcore_axis_name="s"),
        scratch_types=[
            pltpu.VMEM((_NCH, _CHUNK), jnp.int32),
            pltpu.VMEM((_CHUNK, _N_EMBED), jnp.float32),
            pltpu.SemaphoreType.DMA,
        ],
        compiler_params=pltpu.CompilerParams(
            use_tc_tiling_on_sc=False,
            needs_layout_passes=False,
        ),
    )
    out = call(x_flat, W)
    return out.reshape(_BATCH, _HIST, _N_EMBED)


# speedup vs baseline: 1.0540x; 1.0540x over previous
import jax
import jax.numpy as jnp
from jax import lax
from jax.experimental import pallas as pl
from jax.experimental.pallas import tpu as pltpu
from jax.experimental.pallas import tpu_sc as plsc

_N_EMBED = 64
_BATCH = 4096
_HIST = 50

_N_TOT = _BATCH * _HIST
_NW = 32
_PER_W = _N_TOT // _NW
_CHUNK = 128
_NCH = _PER_W // _CHUNK
_SUP = 5
_NS = _NCH // _SUP
_SROWS = _SUP * _CHUNK
_LANES = 16
_GRPS = _SROWS // _LANES


def _zero_fix(idx_v, buf, s):
    acc = idx_v[_SUP * s, pl.ds(0, _LANES)]
    for g in range(1, _GRPS):
        j = _SUP * s + g // 8
        acc = jnp.minimum(acc, idx_v[j, pl.ds((g % 8) * _LANES, _LANES)])
    cnt = plsc.all_reduce_population_count(acc == 0)

    @pl.when(cnt[0] > 0)
    def _():
        zeros = jnp.zeros((_LANES,), jnp.float32)

        def grp(g, carry):
            j = _SUP * s + g // 8
            col = (g % 8) * _LANES
            idxs = idx_v[j, pl.ds(col, _LANES)]
            msk = idxs == 0
            gcnt = plsc.all_reduce_population_count(msk)

            @pl.when(gcnt[0] > 0)
            def _():
                rowpos = g * _LANES + lax.iota(jnp.int32, 16)

                def colloop(c, carry2):
                    colpos = jnp.full((_LANES,), c, jnp.int32)
                    plsc.store_scatter(buf, [rowpos, colpos], zeros,
                                       mask=msk)
                    return carry2

                lax.fori_loop(0, _N_EMBED, colloop, 0)
            return carry

        lax.fori_loop(0, _GRPS, grp, 0)


def _emb_body(x_hbm, w_hbm, out_hbm, idx_v, buf0, buf1, gsem, osem):
    wid = lax.axis_index("s") * 2 + lax.axis_index("c")
    bufs = (buf0, buf1)
    pltpu.sync_copy(x_hbm.at[wid], idx_v)

    def issue_gathers(s):
        b = bufs[s % 2]
        hs = []
        for k in range(_SUP):
            hs.append(pltpu.async_copy(
                w_hbm.at[idx_v.at[_SUP * s + k]],
                b.at[pl.ds(k * _CHUNK, _CHUNK)], gsem))
        return hs

    gh = {0: issue_gathers(0)}
    oh = {}
    for s in range(_NS):
        b = bufs[s % 2]
        if s + 1 < _NS:
            if s - 1 >= 0:
                oh[s - 1].wait()
            gh[s + 1] = issue_gathers(s + 1)
        for h in gh.pop(s):
            h.wait()
        _zero_fix(idx_v, b, s)
        oh[s] = pltpu.async_copy(b, out_hbm.at[wid, s], osem)
    oh[_NS - 2].wait()
    oh[_NS - 1].wait()


@jax.jit
def kernel(x, W):
    x_flat = x.reshape(_NW, _NCH, _CHUNK)
    call = pl.kernel(
        _emb_body,
        out_type=jax.ShapeDtypeStruct((_NW, _NS, _SROWS, _N_EMBED),
                                      jnp.float32),
        mesh=plsc.VectorSubcoreMesh(core_axis_name="c", subcore_axis_name="s"),
        scratch_types=[
            pltpu.VMEM((_NCH, _CHUNK), jnp.int32),
            pltpu.VMEM((_SROWS, _N_EMBED), jnp.float32),
            pltpu.VMEM((_SROWS, _N_EMBED), jnp.float32),
            pltpu.SemaphoreType.DMA,
            pltpu.SemaphoreType.DMA,
        ],
        compiler_params=pltpu.CompilerParams(
            use_tc_tiling_on_sc=False,
            needs_layout_passes=False,
        ),
    )
    out = call(x_flat, W)
    return out.reshape(_BATCH, _HIST, _N_EMBED)
